# baseline (speedup 1.0000x reference)
import jax
import jax.numpy as jnp
from jax.experimental import pallas as pl
from jax.experimental.pallas import tpu as pltpu
from jax.experimental.pallas import tpu_sc as plsc

EMB = 64
NUM_WORKERS = 32
BATCH_BLOCK = 4096


def _sc_gather3(school_table, goal_table, method_table,
                school_idx, goal_idx, method_idx):
    n = school_idx.shape[0]
    b_per_w = n // NUM_WORKERS
    ch = 128
    n_ch = b_per_w // ch
    sidx = school_idx.astype(jnp.float32).reshape(n // ch, ch)
    gidx = goal_idx.astype(jnp.float32).reshape(n // ch, ch)
    midx = method_idx.astype(jnp.float32).reshape(n // ch, ch)
    out_t = jax.ShapeDtypeStruct((n, 128), jnp.float32)
    row_buf = pltpu.VMEM((ch, 128), jnp.float32)
    idx_buf = pltpu.VMEM((ch,), jnp.int32)
    fidx_buf = pltpu.VMEM((ch,), jnp.float32)
    mesh = plsc.VectorSubcoreMesh(core_axis_name="c", subcore_axis_name="s")

    @pl.kernel(
        out_type=[out_t, out_t, out_t], mesh=mesh,
        scratch_types=[
            idx_buf, idx_buf, fidx_buf, fidx_buf, row_buf, row_buf,
            pltpu.SemaphoreType.DMA, pltpu.SemaphoreType.DMA,
            pltpu.SemaphoreType.DMA, pltpu.SemaphoreType.DMA,
        ])
    def gather_kernel(school_hbm, goal_hbm, method_hbm,
                      si_hbm, gi_hbm, mi_hbm,
                      so_hbm, go_hbm, mo_hbm,
                      i0, i1, f0, f1, r0, r1, g0, g1, w0, w1):
        wid = jax.lax.axis_index("s") * 2 + jax.lax.axis_index("c")
        base = wid * b_per_w
        tables = ((school_hbm, si_hbm, so_hbm),
                  (goal_hbm, gi_hbm, go_hbm),
                  (method_hbm, mi_hbm, mo_hbm))
        items = [(t, c) for t in range(3) for c in range(n_ch)]
        ibufs, fbufs = (i0, i1), (f0, f1)
        rbufs, gsems, wsems = (r0, r1), (g0, g1), (w0, w1)
        gathers = [None, None]
        writes = [None, None]
        for k, (t, c) in enumerate(items):
            b = k % 2
            table_hbm, i_hbm, o_hbm = tables[t]
            if writes[b] is not None:
                writes[b].wait()
            pltpu.sync_copy(i_hbm.at[wid * n_ch + c], fbufs[b])
            for j in range(ch // 16):
                s = pl.ds(j * 16, 16)
                ibufs[b][s] = fbufs[b][s].astype(jnp.int32)
            gathers[b] = pltpu.async_copy(
                table_hbm.at[ibufs[b]], rbufs[b], gsems[b])
            pb = 1 - b
            if gathers[pb] is not None:
                gathers[pb].wait()
                pt, pc = items[k - 1]
                writes[pb] = pltpu.async_copy(
                    rbufs[pb],
                    tables[pt][2].at[pl.ds(base + pc * ch, ch)], wsems[pb])
                gathers[pb] = None
        lb = (len(items) - 1) % 2
        gathers[lb].wait()
        lt, lc = items[-1]
        writes[lb] = pltpu.async_copy(
            rbufs[lb], tables[lt][2].at[pl.ds(base + lc * ch, ch)],
            wsems[lb])
        writes[0].wait()
        writes[1].wait()

    return gather_kernel(school_table, goal_table, method_table,
                         sidx, gidx, midx)


def _tower_body(school_ref, goal_ref, method_ref, pca_ref,
                W1x_ref, Wp_ref, b1_ref, W2_ref, b2_ref, W3_ref, b3_ref,
                out_ref):
    f32 = jnp.float32
    bf16 = jnp.bfloat16
    pca = pca_ref[...]
    pc = pca.shape[1]

    def half(ref, c):
        return jnp.where(pca[:, c:c + 1] > 0.5, ref[:, EMB:2 * EMB],
                         ref[:, 0:EMB])

    x = jnp.concatenate(
        [half(school_ref, pc - 3), half(goal_ref, pc - 2),
         half(method_ref, pc - 1)], axis=-1).astype(bf16)
    h = jnp.dot(x, W1x_ref[...], preferred_element_type=f32)
    h += jnp.dot(pca.astype(bf16), Wp_ref[...],
                 preferred_element_type=f32)
    h = jnp.maximum(h + b1_ref[...], 0.0).astype(bf16)
    h = jnp.dot(h, W2_ref[...], preferred_element_type=f32)
    h = jnp.maximum(h + b2_ref[...], 0.0).astype(bf16)
    out_ref[...] = jnp.dot(h, W3_ref[...],
                           preferred_element_type=f32) + b3_ref[...]


def kernel(school_idx, goal_idx, method_idx, subject_pca, grade_pca,
           school_table, goal_table, method_table,
           subject_W, subject_b, grade_W, grade_b,
           W1, b1, W2, b2, W3, b3):
    n = school_idx.shape[0]
    bf16 = jnp.bfloat16
    f32 = jnp.float32
    school_emb, goal_emb, method_emb = _sc_gather3(
        school_table.reshape(-1, 128),
        goal_table.reshape(-1, 128),
        method_table.reshape(-1, 128),
        school_idx // 2, goal_idx // 2, method_idx // 2)

    W1x = W1[0:192].astype(bf16)
    Wp = jnp.concatenate(
        [subject_W @ W1[192:224], grade_W @ W1[224:256],
         jnp.zeros((3, W1.shape[1]), f32)], axis=0).astype(bf16)
    b1f = b1 + subject_b @ W1[192:224] + grade_b @ W1[224:256]
    pca = jnp.concatenate(
        [subject_pca, grade_pca,
         (school_idx % 2)[:, None].astype(f32),
         (goal_idx % 2)[:, None].astype(f32),
         (method_idx % 2)[:, None].astype(f32)], axis=-1)

    bs = BATCH_BLOCK

    def full_spec(shape):
        return pl.BlockSpec(shape, lambda i: (0,) * len(shape))

    def batch_spec(cols):
        return pl.BlockSpec((bs, cols), lambda i: (i, 0))

    out = pl.pallas_call(
        _tower_body,
        grid=(n // bs,),
        in_specs=[
            batch_spec(128), batch_spec(128), batch_spec(128),
            batch_spec(pca.shape[1]),
            full_spec(W1x.shape), full_spec(Wp.shape),
            full_spec((1, b1f.shape[0])),
            full_spec(W2.shape), full_spec((1, b2.shape[0])),
            full_spec(W3.shape), full_spec((1, b3.shape[0])),
        ],
        out_specs=batch_spec(W3.shape[1]),
        out_shape=jax.ShapeDtypeStruct((n, W3.shape[1]), jnp.float32),
    )(school_emb, goal_emb, method_emb, pca,
      W1x, Wp, b1f.reshape(1, -1),
      W2.astype(bf16), b2.reshape(1, -1),
      W3.astype(bf16), b3.reshape(1, -1))
    return out


# speedup vs baseline: 1.1179x; 1.1179x over previous
import jax
import jax.numpy as jnp
from jax.experimental import pallas as pl
from jax.experimental.pallas import tpu as pltpu
from jax.experimental.pallas import tpu_sc as plsc

EMB = 64
NUM_WORKERS = 32
BATCH_BLOCK = 4096


def _sc_gather3(school_table, goal_table, method_table,
                school_idx, goal_idx, method_idx):
    n = school_idx.shape[0]
    b_per_w = n // NUM_WORKERS
    ch = 128
    n_ch = b_per_w // ch
    sidx = school_idx.astype(jnp.float32).reshape(n // ch, ch)
    gidx = goal_idx.astype(jnp.float32).reshape(n // ch, ch)
    midx = method_idx.astype(jnp.float32).reshape(n // ch, ch)
    out_t = jax.ShapeDtypeStruct((n, 128), jnp.float32)
    row_buf = pltpu.VMEM((ch, 128), jnp.float32)
    idx_buf = pltpu.VMEM((ch,), jnp.int32)
    fidx_buf = pltpu.VMEM((ch,), jnp.float32)
    mesh = plsc.VectorSubcoreMesh(core_axis_name="c", subcore_axis_name="s")

    @pl.kernel(
        out_type=[out_t, out_t, out_t], mesh=mesh,
        scratch_types=[
            idx_buf, idx_buf, fidx_buf, fidx_buf, row_buf, row_buf,
            pltpu.SemaphoreType.DMA, pltpu.SemaphoreType.DMA,
            pltpu.SemaphoreType.DMA, pltpu.SemaphoreType.DMA,
        ])
    def gather_kernel(school_hbm, goal_hbm, method_hbm,
                      si_hbm, gi_hbm, mi_hbm,
                      so_hbm, go_hbm, mo_hbm,
                      i0, i1, f0, f1, r0, r1, g0, g1, w0, w1):
        wid = jax.lax.axis_index("s") * 2 + jax.lax.axis_index("c")
        base = wid * b_per_w
        tables = ((school_hbm, si_hbm, so_hbm),
                  (goal_hbm, gi_hbm, go_hbm),
                  (method_hbm, mi_hbm, mo_hbm))
        items = [(t, c) for t in range(3) for c in range(n_ch)]
        ibufs, fbufs = (i0, i1), (f0, f1)
        rbufs, gsems, wsems = (r0, r1), (g0, g1), (w0, w1)
        gathers = [None, None]
        writes = [None, None]
        for k, (t, c) in enumerate(items):
            b = k % 2
            table_hbm, i_hbm, o_hbm = tables[t]
            if writes[b] is not None:
                writes[b].wait()
            pltpu.sync_copy(i_hbm.at[wid * n_ch + c], fbufs[b])
            for j in range(ch // 16):
                s = pl.ds(j * 16, 16)
                ibufs[b][s] = fbufs[b][s].astype(jnp.int32)
            gathers[b] = pltpu.async_copy(
                table_hbm.at[ibufs[b]], rbufs[b], gsems[b])
            pb = 1 - b
            if gathers[pb] is not None:
                gathers[pb].wait()
                pt, pc = items[k - 1]
                writes[pb] = pltpu.async_copy(
                    rbufs[pb],
                    tables[pt][2].at[pl.ds(base + pc * ch, ch)], wsems[pb])
                gathers[pb] = None
        lb = (len(items) - 1) % 2
        gathers[lb].wait()
        lt, lc = items[-1]
        writes[lb] = pltpu.async_copy(
            rbufs[lb], tables[lt][2].at[pl.ds(base + lc * ch, ch)],
            wsems[lb])
        writes[0].wait()
        writes[1].wait()

    return gather_kernel(school_table, goal_table, method_table,
                         sidx, gidx, midx)


def _tower_body(school_ref, goal_ref, method_ref, spca_ref, gpca_ref,
                W1_ref, sW_ref, sb_ref, gW_ref, gb_ref, b1_ref,
                W2_ref, b2_ref, W3_ref, b3_ref, out_ref):
    f32 = jnp.float32
    bf16 = jnp.bfloat16
    x = jnp.concatenate(
        [school_ref[:, 0:EMB], goal_ref[:, 0:EMB], method_ref[:, 0:EMB]],
        axis=-1).astype(bf16)
    h = jnp.dot(x, W1_ref[0:3 * EMB, :].astype(bf16),
                preferred_element_type=f32)
    p1 = jnp.dot(spca_ref[...], sW_ref[...],
                 preferred_element_type=f32) + sb_ref[...]
    p2 = jnp.dot(gpca_ref[...], gW_ref[...],
                 preferred_element_type=f32) + gb_ref[...]
    h += jnp.dot(p1, W1_ref[3 * EMB:3 * EMB + 32, :],
                 preferred_element_type=f32)
    h += jnp.dot(p2, W1_ref[3 * EMB + 32:3 * EMB + 64, :],
                 preferred_element_type=f32)
    h = jnp.maximum(h + b1_ref[...], 0.0).astype(bf16)
    h = jnp.dot(h, W2_ref[...].astype(bf16), preferred_element_type=f32)
    h = jnp.maximum(h + b2_ref[...], 0.0).astype(bf16)
    out_ref[...] = jnp.dot(h, W3_ref[...].astype(bf16),
                           preferred_element_type=f32) + b3_ref[...]


def kernel(school_idx, goal_idx, method_idx, subject_pca, grade_pca,
           school_table, goal_table, method_table,
           subject_W, subject_b, grade_W, grade_b,
           W1, b1, W2, b2, W3, b3):
    n = school_idx.shape[0]
    pad = ((0, 0), (0, 128 - EMB))
    school_emb, goal_emb, method_emb = _sc_gather3(
        jnp.pad(school_table, pad),
        jnp.pad(goal_table, pad),
        jnp.pad(method_table, pad),
        school_idx, goal_idx, method_idx)

    bs = BATCH_BLOCK

    def full_spec(shape):
        return pl.BlockSpec(shape, lambda i: (0,) * len(shape))

    def batch_spec(cols):
        return pl.BlockSpec((bs, cols), lambda i: (i, 0))

    out = pl.pallas_call(
        _tower_body,
        grid=(n // bs,),
        in_specs=[
            batch_spec(128), batch_spec(128), batch_spec(128),
            batch_spec(subject_pca.shape[1]), batch_spec(grade_pca.shape[1]),
            full_spec(W1.shape),
            full_spec(subject_W.shape), full_spec((1, subject_b.shape[0])),
            full_spec(grade_W.shape), full_spec((1, grade_b.shape[0])),
            full_spec((1, b1.shape[0])),
            full_spec(W2.shape), full_spec((1, b2.shape[0])),
            full_spec(W3.shape), full_spec((1, b3.shape[0])),
        ],
        out_specs=batch_spec(W3.shape[1]),
        out_shape=jax.ShapeDtypeStruct((n, W3.shape[1]), jnp.float32),
    )(school_emb, goal_emb, method_emb, subject_pca, grade_pca,
      W1, subject_W, subject_b.reshape(1, -1),
      grade_W, grade_b.reshape(1, -1), b1.reshape(1, -1),
      W2, b2.reshape(1, -1), W3, b3.reshape(1, -1))
    return out


# speedup vs baseline: 1.1443x; 1.0237x over previous
import jax
import jax.numpy as jnp
from jax.experimental import pallas as pl
from jax.experimental.pallas import tpu as pltpu
from jax.experimental.pallas import tpu_sc as plsc

EMB = 64
NUM_WORKERS = 32
BATCH_BLOCK = 4096


def _sc_gather3(school_table, goal_table, method_table,
                school_idx, goal_idx, method_idx):
    n = school_idx.shape[0]
    b_per_w = n // NUM_WORKERS
    ch = 128
    n_ch = b_per_w // ch
    f32 = jnp.float32
    idx_all = (jnp.stack([school_idx, goal_idx, method_idx])
               .astype(f32)
               .reshape(3, NUM_WORKERS, n_ch, ch)
               .transpose(1, 0, 2, 3))
    out_t = jax.ShapeDtypeStruct((n, 128), f32)
    row_buf = pltpu.VMEM((ch, 128), f32)
    idx_buf = pltpu.VMEM((3, n_ch, ch), jnp.int32)
    fidx_buf = pltpu.VMEM((3, n_ch, ch), f32)
    mesh = plsc.VectorSubcoreMesh(core_axis_name="c", subcore_axis_name="s")

    @pl.kernel(
        out_type=[out_t, out_t, out_t], mesh=mesh,
        scratch_types=[
            idx_buf, fidx_buf, row_buf, row_buf,
            pltpu.SemaphoreType.DMA, pltpu.SemaphoreType.DMA,
            pltpu.SemaphoreType.DMA, pltpu.SemaphoreType.DMA,
        ])
    def gather_kernel(school_hbm, goal_hbm, method_hbm, idx_hbm,
                      so_hbm, go_hbm, mo_hbm,
                      ib, fb, r0, r1, g0, g1, w0, w1):
        wid = jax.lax.axis_index("s") * 2 + jax.lax.axis_index("c")
        base = wid * b_per_w
        pltpu.sync_copy(idx_hbm.at[wid], fb)
        for t in range(3):
            for c in range(n_ch):
                for j in range(ch // 16):
                    s = pl.ds(j * 16, 16)
                    ib[t, c, s] = fb[t, c, s].astype(jnp.int32)
        tables = (school_hbm, goal_hbm, method_hbm)
        outs = (so_hbm, go_hbm, mo_hbm)
        items = [(t, c) for t in range(3) for c in range(n_ch)]
        rbufs, gsems, wsems = (r0, r1), (g0, g1), (w0, w1)
        gathers = [None, None]
        writes = [None, None]
        for k, (t, c) in enumerate(items):
            b = k % 2
            if writes[b] is not None:
                writes[b].wait()
            gathers[b] = pltpu.async_copy(
                tables[t].at[ib.at[t, c]], rbufs[b], gsems[b])
            pb = 1 - b
            if gathers[pb] is not None:
                gathers[pb].wait()
                pt, pc = items[k - 1]
                writes[pb] = pltpu.async_copy(
                    rbufs[pb],
                    outs[pt].at[pl.ds(base + pc * ch, ch)], wsems[pb])
                gathers[pb] = None
        lb = (len(items) - 1) % 2
        gathers[lb].wait()
        lt, lc = items[-1]
        writes[lb] = pltpu.async_copy(
            rbufs[lb], outs[lt].at[pl.ds(base + lc * ch, ch)],
            wsems[lb])
        writes[0].wait()
        writes[1].wait()

    return gather_kernel(school_table, goal_table, method_table, idx_all)


def _tower_body(school_ref, goal_ref, method_ref, pca_ref,
                W1x_ref, Wp_ref, b1_ref, W2_ref, b2_ref, W3_ref, b3_ref,
                out_ref):
    f32 = jnp.float32
    bf16 = jnp.bfloat16
    x = jnp.concatenate(
        [school_ref[...], goal_ref[...], method_ref[...]],
        axis=-1).astype(bf16)
    h = jnp.dot(x, W1x_ref[...], preferred_element_type=f32)
    h += jnp.dot(pca_ref[...].astype(bf16), Wp_ref[...],
                 preferred_element_type=f32)
    h = jnp.maximum(h + b1_ref[...], 0.0).astype(bf16)
    h = jnp.dot(h, W2_ref[...], preferred_element_type=f32)
    h = jnp.maximum(h + b2_ref[...], 0.0).astype(bf16)
    out_ref[...] = jnp.dot(h, W3_ref[...],
                           preferred_element_type=f32) + b3_ref[...]


def kernel(school_idx, goal_idx, method_idx, subject_pca, grade_pca,
           school_table, goal_table, method_table,
           subject_W, subject_b, grade_W, grade_b,
           W1, b1, W2, b2, W3, b3):
    n = school_idx.shape[0]
    bf16 = jnp.bfloat16
    pad = ((0, 0), (0, 128 - EMB))
    school_emb, goal_emb, method_emb = _sc_gather3(
        jnp.pad(school_table, pad),
        jnp.pad(goal_table, pad),
        jnp.pad(method_table, pad),
        school_idx, goal_idx, method_idx)

    z = jnp.zeros((128 - EMB, W1.shape[1]), W1.dtype)
    W1x = jnp.concatenate(
        [W1[0:64], z, W1[64:128], z, W1[128:192], z], axis=0).astype(bf16)
    Wp = jnp.concatenate(
        [subject_W @ W1[192:224], grade_W @ W1[224:256]],
        axis=0).astype(bf16)
    b1f = b1 + subject_b @ W1[192:224] + grade_b @ W1[224:256]
    pca = jnp.concatenate([subject_pca, grade_pca], axis=-1)

    bs = BATCH_BLOCK

    def full_spec(shape):
        return pl.BlockSpec(shape, lambda i: (0,) * len(shape))

    def batch_spec(cols):
        return pl.BlockSpec((bs, cols), lambda i: (i, 0))

    out = pl.pallas_call(
        _tower_body,
        grid=(n // bs,),
        in_specs=[
            batch_spec(128), batch_spec(128), batch_spec(128),
            batch_spec(pca.shape[1]),
            full_spec(W1x.shape), full_spec(Wp.shape),
            full_spec((1, b1f.shape[0])),
            full_spec(W2.shape), full_spec((1, b2.shape[0])),
            full_spec(W3.shape), full_spec((1, b3.shape[0])),
        ],
        out_specs=batch_spec(W3.shape[1]),
        out_shape=jax.ShapeDtypeStruct((n, W3.shape[1]), jnp.float32),
    )(school_emb, goal_emb, method_emb, pca,
      W1x, Wp, b1f.reshape(1, -1),
      W2.astype(bf16), b2.reshape(1, -1),
      W3.astype(bf16), b3.reshape(1, -1))
    return out


# speedup vs baseline: 1.1538x; 1.0083x over previous
import jax
import jax.numpy as jnp
from jax.experimental import pallas as pl
from jax.experimental.pallas import tpu as pltpu
from jax.experimental.pallas import tpu_sc as plsc

EMB = 64
NUM_WORKERS = 32
BATCH_BLOCK = 4096


def _sc_gather3(school_table, goal_table, method_table,
                school_idx, goal_idx, method_idx):
    n = school_idx.shape[0]
    b_per_w = n // NUM_WORKERS
    ch = 128
    n_ch = b_per_w // ch
    f32 = jnp.float32
    idx_all = (jnp.stack([school_idx, goal_idx, method_idx])
               .astype(f32)
               .reshape(3, NUM_WORKERS, n_ch, ch)
               .transpose(1, 0, 2, 3))
    out_t = jax.ShapeDtypeStruct((n, 128), f32)
    row_buf = pltpu.VMEM((ch, 128), f32)
    idx_buf = pltpu.VMEM((3, n_ch, ch), jnp.int32)
    fidx_buf = pltpu.VMEM((3, n_ch, ch), f32)
    mesh = plsc.VectorSubcoreMesh(core_axis_name="c", subcore_axis_name="s")

    @pl.kernel(
        out_type=[out_t, out_t, out_t], mesh=mesh,
        scratch_types=[
            idx_buf, fidx_buf, row_buf, row_buf, row_buf, row_buf,
            pltpu.SemaphoreType.DMA, pltpu.SemaphoreType.DMA,
            pltpu.SemaphoreType.DMA, pltpu.SemaphoreType.DMA,
            pltpu.SemaphoreType.DMA, pltpu.SemaphoreType.DMA,
            pltpu.SemaphoreType.DMA, pltpu.SemaphoreType.DMA,
        ])
    def gather_kernel(school_hbm, goal_hbm, method_hbm, idx_hbm,
                      so_hbm, go_hbm, mo_hbm,
                      ib, fb, r0, r1, r2, r3, g0, g1, g2, g3,
                      w0, w1, w2, w3):
        wid = jax.lax.axis_index("s") * 2 + jax.lax.axis_index("c")
        base = wid * b_per_w
        pltpu.sync_copy(idx_hbm.at[wid], fb)
        for t in range(3):
            for c in range(n_ch):
                for j in range(ch // 16):
                    s = pl.ds(j * 16, 16)
                    ib[t, c, s] = fb[t, c, s].astype(jnp.int32)
        tables = (school_hbm, goal_hbm, method_hbm)
        outs = (so_hbm, go_hbm, mo_hbm)
        items = [(t, c) for t in range(3) for c in range(n_ch)]
        D = 4
        rbufs = (r0, r1, r2, r3)
        gsems = (g0, g1, g2, g3)
        wsems = (w0, w1, w2, w3)
        gathers = [None] * D
        writes = [None] * D
        for k in range(len(items) + D - 1):
            if k < len(items):
                t, c = items[k]
                b = k % D
                if writes[b] is not None:
                    writes[b].wait()
                gathers[b] = pltpu.async_copy(
                    tables[t].at[ib.at[t, c]], rbufs[b], gsems[b])
            j = k - (D - 1)
            if 0 <= j < len(items):
                bj = j % D
                gathers[bj].wait()
                jt, jc = items[j]
                writes[bj] = pltpu.async_copy(
                    rbufs[bj],
                    outs[jt].at[pl.ds(base + jc * ch, ch)], wsems[bj])
        for b in range(min(D, len(items))):
            writes[b].wait()

    return gather_kernel(school_table, goal_table, method_table, idx_all)


def _tower_body(school_ref, goal_ref, method_ref, pca_ref,
                W1x_ref, Wp_ref, b1_ref, W2_ref, b2_ref, W3_ref, b3_ref,
                out_ref):
    f32 = jnp.float32
    bf16 = jnp.bfloat16
    x = jnp.concatenate(
        [school_ref[...], goal_ref[...], method_ref[...]],
        axis=-1).astype(bf16)
    h = jnp.dot(x, W1x_ref[...], preferred_element_type=f32)
    h += jnp.dot(pca_ref[...].astype(bf16), Wp_ref[...],
                 preferred_element_type=f32)
    h = jnp.maximum(h + b1_ref[...], 0.0).astype(bf16)
    h = jnp.dot(h, W2_ref[...], preferred_element_type=f32)
    h = jnp.maximum(h + b2_ref[...], 0.0).astype(bf16)
    out_ref[...] = jnp.dot(h, W3_ref[...],
                           preferred_element_type=f32) + b3_ref[...]


def kernel(school_idx, goal_idx, method_idx, subject_pca, grade_pca,
           school_table, goal_table, method_table,
           subject_W, subject_b, grade_W, grade_b,
           W1, b1, W2, b2, W3, b3):
    n = school_idx.shape[0]
    bf16 = jnp.bfloat16
    pad = ((0, 0), (0, 128 - EMB))
    school_emb, goal_emb, method_emb = _sc_gather3(
        jnp.pad(school_table, pad),
        jnp.pad(goal_table, pad),
        jnp.pad(method_table, pad),
        school_idx, goal_idx, method_idx)

    z = jnp.zeros((128 - EMB, W1.shape[1]), W1.dtype)
    W1x = jnp.concatenate(
        [W1[0:64], z, W1[64:128], z, W1[128:192], z], axis=0).astype(bf16)
    Wp = jnp.concatenate(
        [subject_W @ W1[192:224], grade_W @ W1[224:256]],
        axis=0).astype(bf16)
    b1f = b1 + subject_b @ W1[192:224] + grade_b @ W1[224:256]
    pca = jnp.concatenate([subject_pca, grade_pca], axis=-1)

    bs = BATCH_BLOCK

    def full_spec(shape):
        return pl.BlockSpec(shape, lambda i: (0,) * len(shape))

    def batch_spec(cols):
        return pl.BlockSpec((bs, cols), lambda i: (i, 0))

    out = pl.pallas_call(
        _tower_body,
        grid=(n // bs,),
        in_specs=[
            batch_spec(128), batch_spec(128), batch_spec(128),
            batch_spec(pca.shape[1]),
            full_spec(W1x.shape), full_spec(Wp.shape),
            full_spec((1, b1f.shape[0])),
            full_spec(W2.shape), full_spec((1, b2.shape[0])),
            full_spec(W3.shape), full_spec((1, b3.shape[0])),
        ],
        out_specs=batch_spec(W3.shape[1]),
        out_shape=jax.ShapeDtypeStruct((n, W3.shape[1]), jnp.float32),
    )(school_emb, goal_emb, method_emb, pca,
      W1x, Wp, b1f.reshape(1, -1),
      W2.astype(bf16), b2.reshape(1, -1),
      W3.astype(bf16), b3.reshape(1, -1))
    return out
